# Initial kernel scaffold; baseline (speedup 1.0000x reference)
import jax
import jax.numpy as jnp
from jax.experimental import pallas as pl


def kernel(logits, noise):
    raise NotImplementedError("write your pallas kernel here")


# speedup vs baseline: 1.4793x; 1.4793x over previous
import jax
import jax.numpy as jnp
from jax import lax
from jax.experimental import pallas as pl

_EPS = 1e-10
_ROWS = 8


def _fused_body(logits_ref, noise_ref, samples_ref, sel_ref):
    x = logits_ref[...]
    n = noise_ref[...]
    g = -jnp.log(-jnp.log(n + _EPS) + _EPS)
    p = x + g
    pmax = jnp.max(p, axis=-1, keepdims=True)
    iota = lax.broadcasted_iota(jnp.int32, p.shape, 1)
    big = jnp.int32(2**31 - 1)
    idx = jnp.min(jnp.where(p == pmax, iota, big), axis=-1, keepdims=True)
    sel_logit = jnp.max(jnp.where(iota == idx, x, -jnp.inf), axis=-1,
                        keepdims=True)
    m = jnp.max(x, axis=-1, keepdims=True)
    s = jnp.sum(jnp.exp(x - m), axis=-1, keepdims=True)
    samples_ref[...] = idx
    sel_ref[...] = sel_logit - m - jnp.log(s)


def kernel(logits, noise):
    b, v = logits.shape
    samples2, sel2 = pl.pallas_call(
        _fused_body,
        grid=(b // _ROWS,),
        in_specs=[
            pl.BlockSpec((_ROWS, v), lambda i: (i, 0)),
            pl.BlockSpec((_ROWS, v), lambda i: (i, 0)),
        ],
        out_specs=[
            pl.BlockSpec((_ROWS, 1), lambda i: (i, 0)),
            pl.BlockSpec((_ROWS, 1), lambda i: (i, 0)),
        ],
        out_shape=[
            jax.ShapeDtypeStruct((b, 1), jnp.int32),
            jax.ShapeDtypeStruct((b, 1), jnp.float32),
        ],
    )(logits, noise)
    return samples2[:, 0], sel2[:, 0]
